# Initial kernel scaffold; baseline (speedup 1.0000x reference)
import jax
import jax.numpy as jnp
from jax.experimental import pallas as pl


def kernel(x, weight):
    raise NotImplementedError("write your pallas kernel here")


# speedup vs baseline: 1.7555x; 1.7555x over previous
import jax
import jax.numpy as jnp
from jax.experimental import pallas as pl


def _pe_add_body(x_ref, w_ref, o_ref):
    o_ref[...] = x_ref[...] + w_ref[...][None, :, :]


def kernel(x, weight):
    b, n, d = x.shape
    s_blk = 256
    num_blocks = n // s_blk
    return pl.pallas_call(
        _pe_add_body,
        grid=(num_blocks,),
        in_specs=[
            pl.BlockSpec((b, s_blk, d), lambda i: (0, i, 0)),
            pl.BlockSpec((s_blk, d), lambda i: (i, 0)),
        ],
        out_specs=pl.BlockSpec((b, s_blk, d), lambda i: (0, i, 0)),
        out_shape=jax.ShapeDtypeStruct(x.shape, x.dtype),
    )(x, weight[:n])


# speedup vs baseline: 1.8027x; 1.0269x over previous
import jax
import jax.numpy as jnp
from jax.experimental import pallas as pl


def _pe_add_body(x_ref, w_ref, o_ref):
    o_ref[...] = x_ref[...] + w_ref[...][None, :, :]


def kernel(x, weight):
    b, n, d = x.shape
    s_blk = 512
    num_blocks = n // s_blk
    return pl.pallas_call(
        _pe_add_body,
        grid=(num_blocks,),
        in_specs=[
            pl.BlockSpec((b, s_blk, d), lambda i: (0, i, 0)),
            pl.BlockSpec((s_blk, d), lambda i: (i, 0)),
        ],
        out_specs=pl.BlockSpec((b, s_blk, d), lambda i: (0, i, 0)),
        out_shape=jax.ShapeDtypeStruct(x.shape, x.dtype),
    )(x, weight[:n])
